# Initial kernel scaffold; baseline (speedup 1.0000x reference)
import jax
import jax.numpy as jnp
from jax.experimental import pallas as pl


def kernel(x, edge_index, params):
    raise NotImplementedError("write your pallas kernel here")


# speedup vs baseline: 1.7129x; 1.7129x over previous
import functools

import jax
import jax.numpy as jnp
from jax import lax
from jax.experimental import pallas as pl
from jax.experimental.pallas import tpu as pltpu
from jax.experimental.pallas import tpu_sc as plsc

_N = 10000
_E = 320000
_BR = 400
_NBLK = _N // _BR
_B = 80
_NSC = 2
_NT = 16
_EPC = _E // _NSC
_EPT = _EPC // _NT
_NB = _EPT // _B
_NPT = 624
_NREM = _N - _NT * _NPT

_DIMS = [(128, 128), (128, 256), (256, 512), (512, 512), (512, 512)]


def _gather_cat_call(h, src, dst, din):
    mesh = plsc.VectorSubcoreMesh(core_axis_name="c", subcore_axis_name="s")

    @functools.partial(
        pl.kernel,
        out_type=jax.ShapeDtypeStruct((_E, 2 * din), jnp.float32),
        mesh=mesh,
        scratch_types=[
            pltpu.VMEM((_B,), jnp.int32),
            pltpu.VMEM((_B,), jnp.int32),
            pltpu.VMEM((_B, din), jnp.float32),
            pltpu.VMEM((_B, din), jnp.float32),
            pltpu.SemaphoreType.DMA,
            pltpu.SemaphoreType.DMA,
        ],
    )
    def k(h_ref, src_ref, dst_ref, cat_ref, dst_v, src_v, hd_v, hs_v, s1, s2):
        cid = lax.axis_index("c")
        sid = lax.axis_index("s")
        ebase = cid * _EPC + sid * _EPT

        def step(t, carry):
            b0 = ebase + t * _B
            pltpu.sync_copy(dst_ref.at[pl.ds(b0, _B)], dst_v)
            pltpu.sync_copy(src_ref.at[pl.ds(b0, _B)], src_v)
            cp1 = pltpu.async_copy(h_ref.at[dst_v], hd_v, s1)
            cp2 = pltpu.async_copy(h_ref.at[src_v], hs_v, s2)
            cp1.wait()
            cp2.wait()
            pltpu.sync_copy(hd_v, cat_ref.at[pl.ds(b0, _B), pl.ds(0, din)])
            pltpu.sync_copy(hs_v, cat_ref.at[pl.ds(b0, _B), pl.ds(din, din)])
            return carry

        lax.fori_loop(0, _NB, step, 0)

    return k(h, src, dst)


def _call_msg(cat, w1T, b1r, din, dout):
    BE = 800

    def body(c_ref, w_ref, b_ref, m_ref):
        m1 = jnp.dot(c_ref[...], w_ref[...],
                     preferred_element_type=jnp.float32) + b_ref[0, 0]
        m_ref[...] = _bf16_rne(jnp.maximum(m1, 0.0))

    return pl.pallas_call(
        body,
        grid=(_E // BE,),
        in_specs=[
            pl.BlockSpec((BE, 2 * din), lambda i: (i, 0)),
            pl.BlockSpec((2 * din, dout), lambda i: (0, 0)),
            pl.BlockSpec((1, 1, dout), lambda i: (0, 0, 0)),
        ],
        out_specs=pl.BlockSpec((BE, dout), lambda i: (i, 0)),
        out_shape=jax.ShapeDtypeStruct((_E, dout), jnp.float32),
    )(cat, w1T, b1r)


def _segsum_call(m, dst, zeros128, C):
    mesh = plsc.VectorSubcoreMesh(core_axis_name="c", subcore_axis_name="s")

    @functools.partial(
        pl.kernel,
        out_type=jax.ShapeDtypeStruct((2, C, _N, 128), jnp.float32),
        mesh=mesh,
        scratch_types=[
            pltpu.VMEM((_B,), jnp.int32),
            pltpu.VMEM((_B, 128), jnp.float32),
            pltpu.VMEM_SHARED((_N, 128), jnp.float32),
        ],
    )
    def k(m_ref, dst_ref, z_ref, out_ref, dst_v, m_v, acc):
        cid = lax.axis_index("c")
        sid = lax.axis_index("s")
        row0 = sid * _NPT
        ebase = cid * _EPC + sid * _EPT
        for c in range(C):
            pltpu.sync_copy(z_ref.at[pl.ds(row0, _NPT)],
                            acc.at[pl.ds(row0, _NPT)])

            @pl.when(sid == _NT - 1)
            def _():
                pltpu.sync_copy(z_ref.at[pl.ds(_NT * _NPT, _NREM)],
                                acc.at[pl.ds(_NT * _NPT, _NREM)])

            plsc.subcore_barrier()

            def step(t, carry):
                b0 = ebase + t * _B
                pltpu.sync_copy(dst_ref.at[pl.ds(b0, _B)], dst_v)
                pltpu.sync_copy(
                    m_ref.at[pl.ds(b0, _B), pl.ds(c * 128, 128)], m_v)
                pltpu.sync_copy(m_v, acc.at[dst_v], add=True)
                return carry

            lax.fori_loop(0, _NB, step, 0)
            plsc.subcore_barrier()
            pltpu.sync_copy(acc.at[pl.ds(row0, _NPT)],
                            out_ref.at[cid, c, pl.ds(row0, _NPT)])

            @pl.when(sid == _NT - 1)
            def _():
                pltpu.sync_copy(
                    acc.at[pl.ds(_NT * _NPT, _NREM)],
                    out_ref.at[cid, c, pl.ds(_NT * _NPT, _NREM)])

    return k(m, dst, zeros128)


def _deg_call(dst, ones_t, zeros128):
    mesh = plsc.VectorSubcoreMesh(core_axis_name="c", subcore_axis_name="s")

    @functools.partial(
        pl.kernel,
        out_type=jax.ShapeDtypeStruct((2, _N, 128), jnp.float32),
        mesh=mesh,
        scratch_types=[
            pltpu.VMEM((_B,), jnp.int32),
            pltpu.VMEM((_B, 128), jnp.float32),
            pltpu.VMEM_SHARED((_N, 128), jnp.float32),
        ],
    )
    def k(dst_ref, ones_ref, z_ref, out_ref, dst_v, ones_v, acc):
        cid = lax.axis_index("c")
        sid = lax.axis_index("s")
        row0 = sid * _NPT
        ebase = cid * _EPC + sid * _EPT
        pltpu.sync_copy(z_ref.at[pl.ds(row0, _NPT)], acc.at[pl.ds(row0, _NPT)])

        @pl.when(sid == _NT - 1)
        def _():
            pltpu.sync_copy(z_ref.at[pl.ds(_NT * _NPT, _NREM)],
                            acc.at[pl.ds(_NT * _NPT, _NREM)])

        pltpu.sync_copy(ones_ref, ones_v)
        plsc.subcore_barrier()

        def step(t, carry):
            b0 = ebase + t * _B
            pltpu.sync_copy(dst_ref.at[pl.ds(b0, _B)], dst_v)
            pltpu.sync_copy(ones_v, acc.at[dst_v], add=True)
            return carry

        lax.fori_loop(0, _NB, step, 0)
        plsc.subcore_barrier()
        pltpu.sync_copy(acc.at[pl.ds(row0, _NPT)],
                        out_ref.at[cid, pl.ds(row0, _NPT)])

        @pl.when(sid == _NT - 1)
        def _():
            pltpu.sync_copy(acc.at[pl.ds(_NT * _NPT, _NREM)],
                            out_ref.at[cid, pl.ds(_NT * _NPT, _NREM)])

    return k(dst, ones_t, zeros128)


def _bf16_rne(v):
    u = lax.bitcast_convert_type(v, jnp.uint32)
    u = (u + ((u >> jnp.uint32(16)) & jnp.uint32(1))
         + jnp.uint32(0x7FFF)) & jnp.uint32(0xFFFF0000)
    return lax.bitcast_convert_type(u, jnp.float32)


def _call_post(S, degp, h, w2c, uwT, u2T, b3, C, din, dout):
    def body(s_ref, d_ref, h_ref, w2_ref, ua_ref, u2_ref, b3_ref,
             out_ref, st_ref):
        i = pl.program_id(0)
        aggr = jnp.zeros((_BR, dout), jnp.float32)
        for c in range(C):
            sc = s_ref[0, c] + s_ref[1, c]
            w2f = w2_ref[c].astype(jnp.float32)
            hi = _bf16_rne(sc)
            r1 = sc - hi
            mid = _bf16_rne(r1)
            lo = r1 - mid
            aggr = aggr + jnp.dot(hi, w2f,
                                  preferred_element_type=jnp.float32)
            aggr = aggr + jnp.dot(mid, w2f,
                                  preferred_element_type=jnp.float32)
            aggr = aggr + jnp.dot(lo, w2f,
                                  preferred_element_type=jnp.float32)
        deg = d_ref[0, :, 0:1] + d_ref[1, :, 0:1]
        aggr = aggr + deg * b3_ref[0, 0]
        ui = jnp.concatenate([h_ref[...], aggr], axis=1)
        h1 = jnp.dot(ui, ua_ref[...], preferred_element_type=jnp.float32)
        h1 = jnp.maximum(h1 + b3_ref[1, 0], 0.0)
        out = jnp.dot(h1, u2_ref[...],
                      preferred_element_type=jnp.float32) + b3_ref[2, 0]
        out_ref[...] = out

        @pl.when(i == 0)
        def _():
            mu0 = jnp.mean(out, axis=0, keepdims=True)
            d = out - mu0
            st_ref[...] = jnp.concatenate(
                [jnp.sum(d, axis=0, keepdims=True),
                 jnp.sum(d * d, axis=0, keepdims=True),
                 mu0, jnp.zeros((5, dout), jnp.float32)], axis=0)

        @pl.when(i > 0)
        def _():
            mu0 = st_ref[2:3, :]
            d = out - mu0
            st_ref[...] = st_ref[...] + jnp.concatenate(
                [jnp.sum(d, axis=0, keepdims=True),
                 jnp.sum(d * d, axis=0, keepdims=True),
                 jnp.zeros((6, dout), jnp.float32)], axis=0)

    return pl.pallas_call(
        body,
        grid=(_NBLK,),
        in_specs=[
            pl.BlockSpec((2, C, _BR, 128), lambda i: (0, 0, i, 0)),
            pl.BlockSpec((2, _BR, 128), lambda i: (0, i, 0)),
            pl.BlockSpec((_BR, din), lambda i: (i, 0)),
            pl.BlockSpec((C, 128, dout), lambda i: (0, 0, 0)),
            pl.BlockSpec((din + dout, dout), lambda i: (0, 0)),
            pl.BlockSpec((dout, dout), lambda i: (0, 0)),
            pl.BlockSpec((3, 1, dout), lambda i: (0, 0, 0)),
        ],
        out_specs=[
            pl.BlockSpec((_BR, dout), lambda i: (i, 0)),
            pl.BlockSpec((8, dout), lambda i: (0, 0)),
        ],
        out_shape=[
            jax.ShapeDtypeStruct((_N, dout), jnp.float32),
            jax.ShapeDtypeStruct((8, dout), jnp.float32),
        ],
    )(S, degp, h, w2c, uwT, u2T, b3)


def _call_bnres(out, st, h, rwT, pb, din, dout):
    has_rw = rwT is not None

    def body(*refs):
        out_ref, st_ref, h_ref = refs[0], refs[1], refs[2]
        k = 3
        if has_rw:
            rw_ref = refs[k]; k += 1
        pb_ref, hn_ref = refs[k], refs[k + 1]
        dm = st_ref[0] * (1.0 / _N)
        mu = st_ref[2] + dm
        var = st_ref[1] * (1.0 / _N) - dm * dm
        inv = lax.rsqrt(var + 1e-5)
        y = pb_ref[0, 0] * (out_ref[...] - mu) * inv + pb_ref[1, 0]
        if has_rw:
            res = jnp.dot(h_ref[...], rw_ref[...],
                          preferred_element_type=jnp.float32) + pb_ref[2, 0]
        else:
            res = h_ref[...]
        hn_ref[...] = jnp.maximum(y + res, 0.0)

    in_specs = [
        pl.BlockSpec((_BR, dout), lambda i: (i, 0)),
        pl.BlockSpec((8, dout), lambda i: (0, 0)),
        pl.BlockSpec((_BR, din), lambda i: (i, 0)),
    ]
    args = [out, st, h]
    if has_rw:
        in_specs.append(pl.BlockSpec((din, dout), lambda i: (0, 0)))
        args.append(rwT)
    in_specs.append(pl.BlockSpec((3, 1, dout), lambda i: (0, 0, 0)))
    args.append(pb)
    return pl.pallas_call(
        body,
        grid=(_NBLK,),
        in_specs=in_specs,
        out_specs=pl.BlockSpec((_BR, dout), lambda i: (i, 0)),
        out_shape=jax.ShapeDtypeStruct((_N, dout), jnp.float32),
    )(*args)


def _call_final(out, st, h, rwT, pb, lgb, din, dout):
    def body(out_ref, st_ref, h_ref, rw_ref, pb_ref, lgb_ref, y_ref):
        dm = st_ref[0] * (1.0 / _N)
        mu = st_ref[2] + dm
        var = st_ref[1] * (1.0 / _N) - dm * dm
        inv = lax.rsqrt(var + 1e-5)
        y = pb_ref[0, 0] * (out_ref[...] - mu) * inv + pb_ref[1, 0]
        res = jnp.dot(h_ref[...], rw_ref[...],
                      preferred_element_type=jnp.float32) + pb_ref[2, 0]
        hn = jnp.maximum(y + res, 0.0)
        mu2 = jnp.mean(hn, axis=1, keepdims=True)
        d2 = hn - mu2
        var2 = jnp.mean(d2 * d2, axis=1, keepdims=True)
        inv2 = lax.rsqrt(var2 + 1e-5)
        y_ref[...] = lgb_ref[0, 0] * d2 * inv2 + lgb_ref[1, 0]

    return pl.pallas_call(
        body,
        grid=(_NBLK,),
        in_specs=[
            pl.BlockSpec((_BR, dout), lambda i: (i, 0)),
            pl.BlockSpec((8, dout), lambda i: (0, 0)),
            pl.BlockSpec((_BR, din), lambda i: (i, 0)),
            pl.BlockSpec((din, dout), lambda i: (0, 0)),
            pl.BlockSpec((3, 1, dout), lambda i: (0, 0, 0)),
            pl.BlockSpec((2, 1, dout), lambda i: (0, 0, 0)),
        ],
        out_specs=pl.BlockSpec((_BR, dout), lambda i: (i, 0)),
        out_shape=jax.ShapeDtypeStruct((_N, dout), jnp.float32),
    )(out, st, h, rwT, pb, lgb)


def kernel(x, edge_index, params):
    src = edge_index[0]
    dst = edge_index[1]
    zeros128 = jnp.zeros((_N, 128), jnp.float32)
    ones_t = jnp.ones((_B, 128), jnp.float32)

    degp = _deg_call(dst, ones_t, zeros128)

    h = x
    for i, (din, dout) in enumerate(_DIMS, start=1):
        p = params['layer%d' % i]
        C = dout // 128

        cat = _gather_cat_call(h, src, dst, din)
        m = _call_msg(cat, p['mW1'].T, p['mb1'].reshape(1, 1, dout), din, dout)
        S = _segsum_call(m, dst, zeros128, C)

        w2c = p['mW2'].T.astype(jnp.bfloat16).reshape(C, 128, dout)
        uwT = p['uW1'].T
        u2T = p['uW2'].T
        b3 = jnp.stack([p['mb2'], p['ub1'], p['ub2']])[:, None, :]
        out, st = _call_post(S, degp, h, w2c, uwT, u2T, b3, C, din, dout)

        if i < 5:
            rwT = p['rW'].T if i >= 2 else None
            rb = p['rb'] if i >= 2 else jnp.zeros((dout,), jnp.float32)
            pb = jnp.stack([p['bn_g'], p['bn_b'], rb])[:, None, :]
            h = _call_bnres(out, st, h, rwT, pb, din, dout)
        else:
            pb = jnp.stack([p['bn_g'], p['bn_b'], p['rb']])[:, None, :]
            lgb = jnp.stack([params['ln_g'], params['ln_b']])[:, None, :]
            h = _call_final(out, st, h, p['rW'].T, pb, lgb, din, dout)
    return h
